# Initial kernel scaffold; baseline (speedup 1.0000x reference)
import jax
import jax.numpy as jnp
from jax.experimental import pallas as pl


def kernel(hidden_states, classifier_weight, e_score_correction_bias):
    raise NotImplementedError("write your pallas kernel here")


# speedup vs baseline: 2.3630x; 2.3630x over previous
import functools

import jax
import jax.numpy as jnp
from jax.experimental import pallas as pl

HIDDEN = 1024
N_EXP = 128
TOP_K = 8
BT = 512


def _router_block(x_ref, w_ref, b_ref, idx_ref, wgt_ref):
    x = x_ref[...]
    w = w_ref[...]
    logits = jax.lax.dot_general(
        x, w, (((1,), (1,)), ((), ())),
        preferred_element_type=jnp.float32,
    )
    m = jnp.max(logits, axis=-1, keepdims=True)
    e = jnp.exp(logits - m)
    s = jnp.sum(e, axis=-1, keepdims=True)
    scores = e / s
    corrected = scores + b_ref[...]

    lane = jax.lax.broadcasted_iota(jnp.int32, (BT, N_EXP), 1)
    idx_cols = []
    wgt_cols = []
    work = corrected
    for _ in range(TOP_K):
        cur_max = jnp.max(work, axis=-1, keepdims=True)
        is_max = work == cur_max
        pick = jnp.min(jnp.where(is_max, lane, N_EXP), axis=-1, keepdims=True)
        sel = lane == pick
        idx_cols.append(pick)
        wgt_cols.append(jnp.sum(jnp.where(sel, scores, 0.0), axis=-1,
                                keepdims=True))
        work = jnp.where(sel, -jnp.inf, work)
    idx_ref[...] = jnp.concatenate(idx_cols, axis=1)
    wgt_ref[...] = jnp.concatenate(wgt_cols, axis=1)


@functools.partial(jax.jit, static_argnames=("interpret",))
def kernel(hidden_states, classifier_weight, e_score_correction_bias,
           interpret=False):
    n_tokens = hidden_states.shape[0]
    bias2d = e_score_correction_bias.reshape(1, N_EXP)
    grid = (n_tokens // BT,)
    idx, wgt = pl.pallas_call(
        _router_block,
        grid=grid,
        in_specs=[
            pl.BlockSpec((BT, HIDDEN), lambda i: (i, 0)),
            pl.BlockSpec((N_EXP, HIDDEN), lambda i: (0, 0)),
            pl.BlockSpec((1, N_EXP), lambda i: (0, 0)),
        ],
        out_specs=[
            pl.BlockSpec((BT, TOP_K), lambda i: (i, 0)),
            pl.BlockSpec((BT, TOP_K), lambda i: (i, 0)),
        ],
        out_shape=[
            jax.ShapeDtypeStruct((n_tokens, TOP_K), jnp.int32),
            jax.ShapeDtypeStruct((n_tokens, TOP_K), jnp.float32),
        ],
        interpret=interpret,
    )(hidden_states, classifier_weight, bias2d)
    return idx, wgt.astype(hidden_states.dtype)


# speedup vs baseline: 2.4580x; 1.0402x over previous
import functools

import jax
import jax.numpy as jnp
from jax import lax
from jax.experimental import pallas as pl
from jax.experimental.pallas import tpu as pltpu
from jax.experimental.pallas import tpu_sc as plsc

HIDDEN = 1024
N_EXP = 128
TOP_K = 8
BT = 512
N_TOK = 32768
NW = 32
TPW = N_TOK // NW
CHUNK = 512
NCHUNK = TPW // CHUNK


def _scores_block(x_ref, w_ref, s_ref):
    logits = jax.lax.dot_general(
        x_ref[...], w_ref[...], (((1,), (1,)), ((), ())),
        preferred_element_type=jnp.float32,
    )
    m = jnp.max(logits, axis=-1, keepdims=True)
    e = jnp.exp(logits - m)
    s = jnp.sum(e, axis=-1, keepdims=True)
    s_ref[...] = e / s


def _tc_scores(hidden_states, classifier_weight):
    return pl.pallas_call(
        _scores_block,
        grid=(N_TOK // BT,),
        in_specs=[
            pl.BlockSpec((BT, HIDDEN), lambda i: (i, 0)),
            pl.BlockSpec((N_EXP, HIDDEN), lambda i: (0, 0)),
        ],
        out_specs=pl.BlockSpec((BT, N_EXP), lambda i: (i, 0)),
        out_shape=jax.ShapeDtypeStruct((N_TOK, N_EXP), jnp.float32),
    )(hidden_states, classifier_weight)


def _merge(a, b):
    (ka, va), (kb, vb) = a, b
    lane = lax.iota(jnp.int32, 16)
    first8 = lane < 8
    ck = jnp.where(first8, ka, lax.rev(kb, (0,)))
    cv = jnp.where(first8, va, lax.rev(vb, (0,)))
    return plsc.sort_key_val(ck, cv, descending=True)


def _sc_topk_body(scores_hbm, bias_hbm, idx_hbm, wgt_hbm,
                  bias_v, buf_v, oidx_v, owgt_v):
    wid = lax.axis_index("s") * 2 + lax.axis_index("c")
    pltpu.sync_copy(bias_hbm, bias_v)
    lane = lax.iota(jnp.int32, 16)
    first8 = lane < 8

    for c in range(NCHUNK):
        base = wid * TPW + c * CHUNK
        pltpu.sync_copy(scores_hbm.at[pl.ds(base, CHUNK)], buf_v)

        def token_body(r, _):
            groups = []
            for g in range(8):
                k = buf_v[r, pl.ds(g * 16, 16)] + bias_v[pl.ds(g * 16, 16)]
                v = lane + g * 16
                groups.append(plsc.sort_key_val(k, v, descending=True))
            m01 = _merge(groups[0], groups[1])
            m23 = _merge(groups[2], groups[3])
            m45 = _merge(groups[4], groups[5])
            m67 = _merge(groups[6], groups[7])
            fk, fv = _merge(_merge(m01, m23), _merge(m45, m67))
            del fk
            row = jnp.full((16,), r, dtype=jnp.int32)
            w = plsc.load_gather(buf_v, [row, fv])
            off = pl.multiple_of(r * 8, 8)
            plsc.store_compressed(oidx_v.at[pl.ds(off, 16)], fv, mask=first8)
            plsc.store_compressed(owgt_v.at[pl.ds(off, 16)], w, mask=first8)
            return 0

        lax.fori_loop(0, CHUNK, token_body, 0)
        pltpu.sync_copy(oidx_v.at[pl.ds(0, CHUNK * TOP_K)],
                        idx_hbm.at[pl.ds(base * TOP_K, CHUNK * TOP_K)])
        pltpu.sync_copy(owgt_v.at[pl.ds(0, CHUNK * TOP_K)],
                        wgt_hbm.at[pl.ds(base * TOP_K, CHUNK * TOP_K)])


def _sc_topk(scores, bias):
    mesh = plsc.VectorSubcoreMesh(core_axis_name="c", subcore_axis_name="s")
    return pl.kernel(
        _sc_topk_body,
        out_type=[
            jax.ShapeDtypeStruct((N_TOK * TOP_K,), jnp.int32),
            jax.ShapeDtypeStruct((N_TOK * TOP_K,), jnp.float32),
        ],
        mesh=mesh,
        compiler_params=pltpu.CompilerParams(needs_layout_passes=False),
        scratch_types=[
            pltpu.VMEM((N_EXP,), jnp.float32),
            pltpu.VMEM((CHUNK, N_EXP), jnp.float32),
            pltpu.VMEM((CHUNK * TOP_K + 8,), jnp.int32),
            pltpu.VMEM((CHUNK * TOP_K + 8,), jnp.float32),
        ],
    )(scores, bias)


@jax.jit
def kernel(hidden_states, classifier_weight, e_score_correction_bias):
    scores = _tc_scores(hidden_states, classifier_weight)
    idx, wgt = _sc_topk(scores, e_score_correction_bias)
    return (idx.reshape(N_TOK, TOP_K),
            wgt.reshape(N_TOK, TOP_K).astype(hidden_states.dtype))


# speedup vs baseline: 2.6029x; 1.0590x over previous
import functools

import jax
import jax.numpy as jnp
from jax import lax
from jax.experimental import pallas as pl
from jax.experimental.pallas import tpu as pltpu
from jax.experimental.pallas import tpu_sc as plsc

HIDDEN = 1024
N_EXP = 128
TOP_K = 8
BT = 512
N_TOK = 32768
NW = 32
TPW = N_TOK // NW
CHUNK = 256
NCHUNK = TPW // CHUNK


def _scores_block(x_ref, w_ref, s_ref):
    logits = jax.lax.dot_general(
        x_ref[...], w_ref[...], (((1,), (1,)), ((), ())),
        preferred_element_type=jnp.float32,
    )
    m = jnp.max(logits, axis=-1, keepdims=True)
    e = jnp.exp(logits - m)
    s = jnp.sum(e, axis=-1, keepdims=True)
    s_ref[...] = e / s


def _tc_scores(hidden_states, classifier_weight):
    return pl.pallas_call(
        _scores_block,
        grid=(N_TOK // BT,),
        in_specs=[
            pl.BlockSpec((BT, HIDDEN), lambda i: (i, 0)),
            pl.BlockSpec((N_EXP, HIDDEN), lambda i: (0, 0)),
        ],
        out_specs=pl.BlockSpec((BT, N_EXP), lambda i: (i, 0)),
        out_shape=jax.ShapeDtypeStruct((N_TOK, N_EXP), jnp.float32),
    )(hidden_states, classifier_weight)


def _merge(a, b):
    (ka, va), (kb, vb) = a, b
    lane = lax.iota(jnp.int32, 16)
    first8 = lane < 8
    ck = jnp.where(first8, ka, lax.rev(kb, (0,)))
    cv = jnp.where(first8, va, lax.rev(vb, (0,)))
    return plsc.sort_key_val(ck, cv, descending=True)


def _sc_topk_body(scores_hbm, bias_hbm, idx_hbm, wgt_hbm,
                  bias_v, buf_v, oidx_v, owgt_v):
    wid = lax.axis_index("s") * 2 + lax.axis_index("c")
    pltpu.sync_copy(bias_hbm, bias_v)
    lane = lax.iota(jnp.int32, 16)
    first8 = lane < 8

    for c in range(NCHUNK):
        base = wid * TPW + c * CHUNK
        pltpu.sync_copy(scores_hbm.at[pl.ds(base, CHUNK)], buf_v)

        def token_body(r, _):
            groups = []
            for g in range(8):
                k = buf_v[r, pl.ds(g * 16, 16)] + bias_v[pl.ds(g * 16, 16)]
                v = lane + g * 16
                groups.append(plsc.sort_key_val(k, v, descending=True))
            m01 = _merge(groups[0], groups[1])
            m23 = _merge(groups[2], groups[3])
            m45 = _merge(groups[4], groups[5])
            m67 = _merge(groups[6], groups[7])
            fk, fv = _merge(_merge(m01, m23), _merge(m45, m67))
            del fk
            row = jnp.full((16,), r, dtype=jnp.int32)
            w = plsc.load_gather(buf_v, [row, fv])
            plsc.store_scatter(oidx_v, [row, lane], fv, mask=first8)
            plsc.store_scatter(owgt_v, [row, lane], w, mask=first8)
            return 0

        lax.fori_loop(0, CHUNK, token_body, 0)
        pltpu.sync_copy(oidx_v, idx_hbm.at[pl.ds(base, CHUNK)])
        pltpu.sync_copy(owgt_v, wgt_hbm.at[pl.ds(base, CHUNK)])


def _sc_topk(scores, bias):
    mesh = plsc.VectorSubcoreMesh(core_axis_name="c", subcore_axis_name="s")
    return pl.kernel(
        _sc_topk_body,
        out_type=[
            jax.ShapeDtypeStruct((N_TOK, TOP_K), jnp.int32),
            jax.ShapeDtypeStruct((N_TOK, TOP_K), jnp.float32),
        ],
        mesh=mesh,
        compiler_params=pltpu.CompilerParams(needs_layout_passes=False),
        scratch_types=[
            pltpu.VMEM((N_EXP,), jnp.float32),
            pltpu.VMEM((CHUNK, N_EXP), jnp.float32),
            pltpu.VMEM((CHUNK, TOP_K), jnp.int32),
            pltpu.VMEM((CHUNK, TOP_K), jnp.float32),
        ],
    )(scores, bias)


@jax.jit
def kernel(hidden_states, classifier_weight, e_score_correction_bias):
    scores = _tc_scores(hidden_states, classifier_weight)
    idx, wgt = _sc_topk(scores, e_score_correction_bias)
    return idx, wgt.astype(hidden_states.dtype)
